# baseline (speedup 1.0000x reference)
import functools

import jax
import jax.numpy as jnp
from jax import lax
from jax.experimental import pallas as pl
from jax.experimental.pallas import tpu as pltpu
from jax.experimental.pallas import tpu_sc as plsc

N = 10000
S = 128
HID = 16
OUT = 3
C = 1152
NBUF = 2
NTILES = 32
RPT = 4
NVEC = N // 16
GPC = C // 16
CW = 4 * C

_mesh = plsc.VectorSubcoreMesh(core_axis_name="c", subcore_axis_name="s")


def _make_conv(F_in, E_pad):
    npasses = F_in // (NTILES * RPT)
    nchunks = E_pad // C
    niters = nchunks // NBUF

    @functools.partial(
        pl.kernel,
        out_type=jax.ShapeDtypeStruct((2 * F_in, N), jnp.float32),
        mesh=_mesh,
        compiler_params=pltpu.CompilerParams(needs_layout_passes=False),
        scratch_types=[
            pltpu.VMEM((RPT, N), jnp.float32),
            pltpu.VMEM((2 * RPT, N), jnp.float32),
            pltpu.VMEM((NBUF, C), jnp.int32),
            pltpu.VMEM((NBUF, C), jnp.int32),
            pltpu.VMEM((NBUF, C), jnp.float32),
            pltpu.VMEM((NBUF, C), jnp.float32),
            pltpu.SemaphoreType.DMA,
            pltpu.SemaphoreType.DMA,
            pltpu.SemaphoreType.DMA,
        ],
    )
    def conv(h_hbm, src_hbm, dst_hbm, w0_hbm, w1_hbm, out_hbm,
             rows_v, acc_v, src_v, dst_v, w0_v, w1_v, sem0, sem1, semf):
        cid = lax.axis_index("c")
        sid = lax.axis_index("s")
        wid = sid * 2 + cid
        sems = (sem0, sem1)

        def start(c, b):
            pltpu.async_copy(src_hbm.at[pl.ds(c * C, C)], src_v.at[b], sems[b])
            pltpu.async_copy(dst_hbm.at[pl.ds(c * C, C)], dst_v.at[b], sems[b])
            pltpu.async_copy(w0_hbm.at[pl.ds(c * C, C)], w0_v.at[b], sems[b])
            pltpu.async_copy(w1_hbm.at[pl.ds(c * C, C)], w1_v.at[b], sems[b])

        def wait(c, b):
            pltpu.make_async_copy(src_hbm.at[pl.ds(c * C, C)], src_v.at[b], sems[b]).wait()
            pltpu.make_async_copy(dst_hbm.at[pl.ds(c * C, C)], dst_v.at[b], sems[b]).wait()
            pltpu.make_async_copy(w0_hbm.at[pl.ds(c * C, C)], w0_v.at[b], sems[b]).wait()
            pltpu.make_async_copy(w1_hbm.at[pl.ds(c * C, C)], w1_v.at[b], sems[b]).wait()

        def process(b):
            @plsc.parallel_loop(0, GPC, 1, unroll=4)
            def group(g):
                s16 = src_v[b, pl.ds(g * 16, 16)]
                d16 = dst_v[b, pl.ds(g * 16, 16)]
                v0 = w0_v[b, pl.ds(g * 16, 16)]
                v1 = w1_v[b, pl.ds(g * 16, 16)]
                for j in range(RPT):
                    jv = jnp.full((16,), j, jnp.int32)
                    g16 = plsc.load_gather(rows_v, [jv, s16])
                    plsc.addupdate_scatter(
                        acc_v, [jnp.full((16,), j, jnp.int32), d16], g16 * v0)
                    plsc.addupdate_scatter(
                        acc_v, [jnp.full((16,), RPT + j, jnp.int32), d16], g16 * v1)

        for p in range(npasses):
            f_base = p * (NTILES * RPT) + wid * RPT

            for j in range(RPT):
                pltpu.async_copy(h_hbm.at[f_base + j], rows_v.at[j], semf)

            @plsc.parallel_loop(0, NVEC, 1, unroll=8)
            def zero(i):
                z = jnp.zeros((16,), jnp.float32)
                for r in range(2 * RPT):
                    acc_v[r, pl.ds(i * 16, 16)] = z

            for j in range(RPT):
                pltpu.make_async_copy(
                    h_hbm.at[f_base + j], rows_v.at[j], semf).wait()

            for b in range(NBUF):
                start(b, b)

            def body(i, _):
                for b in range(NBUF):
                    c = NBUF * i + b
                    wait(c, b)
                    process(b)

                    @pl.when(i < niters - 1)
                    def _():
                        start(c + NBUF, b)

                return 0

            lax.fori_loop(0, niters, body, 0)

            for j in range(RPT):
                pltpu.async_copy(acc_v.at[j], out_hbm.at[f_base + j], semf)
                pltpu.async_copy(
                    acc_v.at[RPT + j], out_hbm.at[F_in + f_base + j], semf)
            for j in range(RPT):
                pltpu.make_async_copy(
                    acc_v.at[j], out_hbm.at[f_base + j], semf).wait()
                pltpu.make_async_copy(
                    acc_v.at[RPT + j], out_hbm.at[F_in + f_base + j], semf).wait()

    return conv


def _mlp_body(x_ref, h1_ref, h2_ref, w1a, w1b, w1c, b1_ref, w2, b2_ref, o_ref):
    acc = jnp.dot(x_ref[...], w1a[...], preferred_element_type=jnp.float32)
    acc = acc + lax.dot_general(
        h1_ref[...], w1b[...], (((0,), (0,)), ((), ())),
        preferred_element_type=jnp.float32)
    acc = acc + lax.dot_general(
        h2_ref[...], w1c[...], (((0,), (0,)), ((), ())),
        preferred_element_type=jnp.float32)
    acc = acc + b1_ref[...]
    acc = jnp.maximum(acc, 0.0)
    o_ref[...] = jnp.dot(acc, w2[...], preferred_element_type=jnp.float32) + b2_ref[...]


_mlp = pl.pallas_call(
    _mlp_body,
    out_shape=jax.ShapeDtypeStruct((N, OUT), jnp.float32),
)


def kernel(x, edge_index, kernel_vals, W1, b1, W2, b2):
    E = edge_index.shape[1]
    pad = (-E) % (NBUF * C)
    src = edge_index[0].astype(jnp.int32)
    dst = edge_index[1].astype(jnp.int32)
    w0 = kernel_vals[0]
    w1v = kernel_vals[1]
    if pad:
        zi = jnp.zeros((pad,), jnp.int32)
        zf = jnp.zeros((pad,), jnp.float32)
        src = jnp.concatenate([src, zi])
        dst = jnp.concatenate([dst, zi])
        w0 = jnp.concatenate([w0, zf])
        w1v = jnp.concatenate([w1v, zf])
    E_pad = E + pad

    conv1 = _make_conv(S, E_pad)
    conv2 = _make_conv(2 * S, E_pad)

    xT = x.T
    h1T = conv1(xT, src, dst, w0, w1v)
    h2T = conv2(h1T, src, dst, w0, w1v)
    return _mlp(x, h1T, h2T, W1[:S], W1[S:3 * S], W1[3 * S:], b1[None, :],
                W2, b2[None, :])


# speedup vs baseline: 1.0512x; 1.0512x over previous
import functools

import jax
import jax.numpy as jnp
from jax import lax
from jax.experimental import pallas as pl
from jax.experimental.pallas import tpu as pltpu
from jax.experimental.pallas import tpu_sc as plsc

N = 10000
S = 128
HID = 16
OUT = 3
C = 640
NBUF = 2
NTILES = 32
RPT = 4
NVEC = N // 16
GPC = C // 16
CW = 4 * C

_mesh = plsc.VectorSubcoreMesh(core_axis_name="c", subcore_axis_name="s")


def _make_conv(F_in, E_pad):
    npasses = F_in // (NTILES * RPT)
    nchunks = E_pad // C
    niters = nchunks // NBUF

    @functools.partial(
        pl.kernel,
        out_type=jax.ShapeDtypeStruct((2 * F_in, N), jnp.float32),
        mesh=_mesh,
        compiler_params=pltpu.CompilerParams(needs_layout_passes=False),
        scratch_types=[
            pltpu.VMEM((RPT, N), jnp.float32),
            pltpu.VMEM((2 * RPT, N), jnp.float32),
            pltpu.VMEM((NBUF, C), jnp.int32),
            pltpu.VMEM((NBUF, C), jnp.float32),
            pltpu.VMEM((NBUF, C), jnp.float32),
            pltpu.SemaphoreType.DMA,
            pltpu.SemaphoreType.DMA,
            pltpu.SemaphoreType.DMA,
        ],
    )
    def conv(h_hbm, sd_hbm, w0_hbm, w1_hbm, out_hbm,
             rows_v, acc_v, sd_v, w0_v, w1_v, sem0, sem1, semf):
        cid = lax.axis_index("c")
        sid = lax.axis_index("s")
        wid = sid * 2 + cid
        sems = (sem0, sem1)

        def start(c, b):
            pltpu.async_copy(sd_hbm.at[pl.ds(c * C, C)], sd_v.at[b], sems[b])
            pltpu.async_copy(w0_hbm.at[pl.ds(c * C, C)], w0_v.at[b], sems[b])
            pltpu.async_copy(w1_hbm.at[pl.ds(c * C, C)], w1_v.at[b], sems[b])

        def wait(c, b):
            pltpu.make_async_copy(sd_hbm.at[pl.ds(c * C, C)], sd_v.at[b], sems[b]).wait()
            pltpu.make_async_copy(w0_hbm.at[pl.ds(c * C, C)], w0_v.at[b], sems[b]).wait()
            pltpu.make_async_copy(w1_hbm.at[pl.ds(c * C, C)], w1_v.at[b], sems[b]).wait()

        def process(b):
            @plsc.parallel_loop(0, GPC, 1, unroll=4)
            def group(g):
                sd16 = sd_v[b, pl.ds(g * 16, 16)]
                s16 = sd16 & jnp.int32(0xFFFF)
                d16 = lax.shift_right_logical(sd16, jnp.full((16,), 16, jnp.int32))
                v0 = w0_v[b, pl.ds(g * 16, 16)]
                v1 = w1_v[b, pl.ds(g * 16, 16)]
                for j in range(RPT):
                    jv = jnp.full((16,), j, jnp.int32)
                    g16 = plsc.load_gather(rows_v, [jv, s16])
                    plsc.addupdate_scatter(
                        acc_v, [jnp.full((16,), j, jnp.int32), d16], g16 * v0)
                    plsc.addupdate_scatter(
                        acc_v, [jnp.full((16,), RPT + j, jnp.int32), d16], g16 * v1)

        for p in range(npasses):
            f_base = p * (NTILES * RPT) + wid * RPT

            for j in range(RPT):
                pltpu.async_copy(h_hbm.at[f_base + j], rows_v.at[j], semf)

            @plsc.parallel_loop(0, NVEC, 1, unroll=8)
            def zero(i):
                z = jnp.zeros((16,), jnp.float32)
                for r in range(2 * RPT):
                    acc_v[r, pl.ds(i * 16, 16)] = z

            for j in range(RPT):
                pltpu.make_async_copy(
                    h_hbm.at[f_base + j], rows_v.at[j], semf).wait()

            for b in range(NBUF):
                start(b, b)

            def body(i, _):
                for b in range(NBUF):
                    c = NBUF * i + b
                    wait(c, b)
                    process(b)

                    @pl.when(i < niters - 1)
                    def _():
                        start(c + NBUF, b)

                return 0

            lax.fori_loop(0, niters, body, 0)

            for j in range(RPT):
                pltpu.async_copy(acc_v.at[j], out_hbm.at[f_base + j], semf)
                pltpu.async_copy(
                    acc_v.at[RPT + j], out_hbm.at[F_in + f_base + j], semf)
            for j in range(RPT):
                pltpu.make_async_copy(
                    acc_v.at[j], out_hbm.at[f_base + j], semf).wait()
                pltpu.make_async_copy(
                    acc_v.at[RPT + j], out_hbm.at[F_in + f_base + j], semf).wait()

    return conv


def _mlp_body(x_ref, h1_ref, h2_ref, w1a, w1b, w1c, b1_ref, w2, b2_ref, o_ref):
    acc = jnp.dot(x_ref[...], w1a[...], preferred_element_type=jnp.float32)
    acc = acc + lax.dot_general(
        h1_ref[...], w1b[...], (((0,), (0,)), ((), ())),
        preferred_element_type=jnp.float32)
    acc = acc + lax.dot_general(
        h2_ref[...], w1c[...], (((0,), (0,)), ((), ())),
        preferred_element_type=jnp.float32)
    acc = acc + b1_ref[...]
    acc = jnp.maximum(acc, 0.0)
    o_ref[...] = jnp.dot(acc, w2[...], preferred_element_type=jnp.float32) + b2_ref[...]


_mlp = pl.pallas_call(
    _mlp_body,
    out_shape=jax.ShapeDtypeStruct((N, OUT), jnp.float32),
)


def kernel(x, edge_index, kernel_vals, W1, b1, W2, b2):
    E = edge_index.shape[1]
    pad = (-E) % (NBUF * C)
    src = edge_index[0].astype(jnp.int32)
    dst = edge_index[1].astype(jnp.int32)
    sd = src | (dst << 16)
    w0 = kernel_vals[0]
    w1v = kernel_vals[1]
    if pad:
        zi = jnp.zeros((pad,), jnp.int32)
        zf = jnp.zeros((pad,), jnp.float32)
        sd = jnp.concatenate([sd, zi])
        w0 = jnp.concatenate([w0, zf])
        w1v = jnp.concatenate([w1v, zf])
    E_pad = E + pad

    conv1 = _make_conv(S, E_pad)
    conv2 = _make_conv(2 * S, E_pad)

    xT = x.T
    h1T = conv1(xT, sd, w0, w1v)
    h2T = conv2(h1T, sd, w0, w1v)
    return _mlp(x, h1T, h2T, W1[:S], W1[S:3 * S], W1[3 * S:], b1[None, :],
                W2, b2[None, :])
